# Initial kernel scaffold; baseline (speedup 1.0000x reference)
import jax
import jax.numpy as jnp
from jax.experimental import pallas as pl


def kernel(constraint_features, edge_index, edge_attr, variable_features, params):
    raise NotImplementedError("write your pallas kernel here")


# speedup vs baseline: 4.7794x; 4.7794x over previous
import functools

import jax
import jax.numpy as jnp
from jax import lax
from jax.experimental import pallas as pl
from jax.experimental.pallas import tpu as pltpu
from jax.experimental.pallas import tpu_sc as plsc

EMB = 64
HALF = 32
N = 25000
E = 800000
N_PAD = 25088
E_PAD = 802816
NC, NS = 2, 16
EROWS = E_PAD // 128
CPT = EROWS // NS
G = 4
GPT = CPT // G
ROWS_PER_TILE = N_PAD // NS
ZROWS = 98

BLK = 512
GRID = N_PAD // BLK


def _dotT(x, w):
    return lax.dot_general(x, w, (((1,), (1,)), ((), ())),
                           precision=lax.Precision.HIGHEST,
                           preferred_element_type=jnp.float32)


def _relu(x):
    return jnp.maximum(x, 0.0)


def _row_spec(d):
    return pl.BlockSpec((BLK, d), lambda i: (i, 0))


def _w_spec(shape):
    return pl.BlockSpec(shape, lambda i: tuple(0 for _ in shape))


def _t1_body(cf, vf, Wce1, bce1, Wce2, bce2, Wve1, bve1, Wve2, bve2,
             vcWl, vcbl, vcWr, cvWl, cvbl,
             c0o, v0o, avco, bvco, acvo):
    c0 = _relu(_dotT(_relu(_dotT(cf[...], Wce1[...]) + bce1[...]), Wce2[...]) + bce2[...])
    v0 = _relu(_dotT(_relu(_dotT(vf[...], Wve1[...]) + bve1[...]), Wve2[...]) + bve2[...])
    c0o[...] = c0
    v0o[...] = v0
    avco[...] = _dotT(c0, vcWl[...]) + vcbl[...]
    bvco[...] = _dotT(v0, vcWr[...])
    acvo[...] = _dotT(v0, cvWl[...]) + cvbl[...]


def _t1(cf, vf, p):
    w = [p['Wce1'], p['bce1'].reshape(1, -1), p['Wce2'], p['bce2'].reshape(1, -1),
         p['Wve1'], p['bve1'].reshape(1, -1), p['Wve2'], p['bve2'].reshape(1, -1),
         p['vc_Wl'], p['vc_bl'].reshape(1, -1), p['vc_Wr'],
         p['cv_Wl'], p['cv_bl'].reshape(1, -1)]
    out_shape = [jax.ShapeDtypeStruct((N_PAD, EMB), jnp.float32)] * 5
    return pl.pallas_call(
        _t1_body,
        grid=(GRID,),
        in_specs=[_row_spec(cf.shape[1]), _row_spec(vf.shape[1])] + [_w_spec(x.shape) for x in w],
        out_specs=[_row_spec(EMB)] * 5,
        out_shape=out_shape,
    )(cf, vf, *w)


def _t2_body(slo, shi, deg, c0, Wfl, Wfh, bf, Wo1a, Wo1b, bo1, Wo2, bo2, cvWr,
             bcvo):
    agg = (_dotT(slo[...], Wfl[...]) + _dotT(shi[...], Wfh[...])
           + deg[:, :1] * bf[...])
    t = _relu(_dotT(agg, Wo1a[...]) + _dotT(c0[...], Wo1b[...]) + bo1[...])
    c1 = _dotT(t, Wo2[...]) + bo2[...]
    bcvo[...] = _dotT(c1, cvWr[...])


def _t2(slo, shi, deg, c0, p):
    w = [p['vc_Wf'][:, :HALF], p['vc_Wf'][:, HALF:], p['vc_bf'].reshape(1, -1),
         p['vc_Wo1'][:, :EMB], p['vc_Wo1'][:, EMB:], p['vc_bo1'].reshape(1, -1),
         p['vc_Wo2'], p['vc_bo2'].reshape(1, -1), p['cv_Wr']]
    return pl.pallas_call(
        _t2_body,
        grid=(GRID,),
        in_specs=[_row_spec(HALF), _row_spec(HALF), _row_spec(16), _row_spec(EMB)]
                 + [_w_spec(x.shape) for x in w],
        out_specs=_row_spec(EMB),
        out_shape=jax.ShapeDtypeStruct((N_PAD, EMB), jnp.float32),
    )(slo, shi, deg, c0, *w)


def _t3_body(slo, shi, deg, v0, Wfl, Wfh, bf, Wo1a, Wo1b, bo1, Wo2, bo2,
             Wout1, bout1, Wout2, Wv1, bv1, Wv2, valo, lgo):
    agg = (_dotT(slo[...], Wfl[...]) + _dotT(shi[...], Wfh[...])
           + deg[:, :1] * bf[...])
    t = _relu(_dotT(agg, Wo1a[...]) + _dotT(v0[...], Wo1b[...]) + bo1[...])
    v1 = _dotT(t, Wo2[...]) + bo2[...]
    lgo[...] = _dotT(_relu(_dotT(v1, Wout1[...]) + bout1[...]), Wout2[...])
    valo[...] = _dotT(_relu(_dotT(v1, Wv1[...]) + bv1[...]), Wv2[...])


def _t3(slo, shi, deg, v0, p):
    w = [p['cv_Wf'][:, :HALF], p['cv_Wf'][:, HALF:], p['cv_bf'].reshape(1, -1),
         p['cv_Wo1'][:, :EMB], p['cv_Wo1'][:, EMB:], p['cv_bo1'].reshape(1, -1),
         p['cv_Wo2'], p['cv_bo2'].reshape(1, -1),
         p['Wout1'], p['bout1'].reshape(1, -1), p['Wout2'],
         p['Wv1'], p['bv1'].reshape(1, -1), p['Wv2']]
    return pl.pallas_call(
        _t3_body,
        grid=(GRID,),
        in_specs=[_row_spec(HALF), _row_spec(HALF), _row_spec(16), _row_spec(EMB)]
                 + [_w_spec(x.shape) for x in w],
        out_specs=[_row_spec(1), _row_spec(1)],
        out_shape=[jax.ShapeDtypeStruct((N_PAD, 1), jnp.float32),
                   jax.ShapeDtypeStruct((N_PAD, 1), jnp.float32)],
    )(slo, shi, deg, v0, *w)


@functools.partial(
    pl.kernel,
    out_type=[jax.ShapeDtypeStruct((NC, N_PAD, HALF), jnp.float32),
              jax.ShapeDtypeStruct((N_PAD, 16), jnp.float32)],
    mesh=plsc.VectorSubcoreMesh(core_axis_name="c", subcore_axis_name="s"),
    compiler_params=pltpu.CompilerParams(use_tc_tiling_on_sc=False),
    scratch_types=[
        pltpu.VMEM((G, 128), jnp.int32),
        pltpu.VMEM((G, 128), jnp.int32),
        pltpu.VMEM((G, 128), jnp.int32),
        pltpu.VMEM((G, 128), jnp.float32),
        pltpu.VMEM((128, HALF), jnp.float32),
        pltpu.VMEM((128, HALF), jnp.float32),
        pltpu.VMEM((128, HALF), jnp.float32),
        pltpu.VMEM((128, 16), jnp.float32),
        pltpu.VMEM((EMB,), jnp.float32),
        pltpu.VMEM_SHARED((N_PAD, HALF), jnp.float32),
        pltpu.VMEM_SHARED((N_PAD, 16), jnp.float32),
        pltpu.SemaphoreType.DMA,
        pltpu.SemaphoreType.DMA,
    ],
)
def _edge_pass(a_hbm, b_hbm, dst_hbm, src_hbm, ea_hbm, we_hbm,
               out_hbm, deg_hbm,
               dstb, dgb, sgb, eab, abuf, bbuf, obuf, dbuf, webuf,
               acc, dacc, sem_a, sem_b):
    cid = lax.axis_index("c")
    sid = lax.axis_index("s")

    pltpu.sync_copy(we_hbm, webuf)

    zero16 = jnp.zeros((16,), jnp.float32)

    def _zrow(i, carry):
        for r in range(HALF // 16):
            obuf[i, pl.ds(r * 16, 16)] = zero16
        dbuf[i, pl.ds(0, 16)] = zero16
        return carry
    lax.fori_loop(0, 128, _zrow, 0)

    def _zstripe(k, carry):
        row = sid * ROWS_PER_TILE + k * ZROWS
        pltpu.sync_copy(obuf.at[pl.ds(0, ZROWS)], acc.at[pl.ds(row, ZROWS)])
        pltpu.sync_copy(dbuf.at[pl.ds(0, ZROWS)], dacc.at[pl.ds(row, ZROWS)])
        return carry
    lax.fori_loop(0, ROWS_PER_TILE // ZROWS, _zstripe, 0)

    onehot = jnp.where(lax.iota(jnp.int32, 16) == 0, 1.0, 0.0)

    def _orow(i, carry):
        dbuf[i, pl.ds(0, 16)] = onehot
        return carry
    lax.fori_loop(0, 128, _orow, 0)
    plsc.subcore_barrier()

    w0 = jnp.where(cid == 0, webuf[pl.ds(0, 16)], webuf[pl.ds(32, 16)])
    w1 = jnp.where(cid == 0, webuf[pl.ds(16, 16)], webuf[pl.ds(48, 16)])
    wes = (w0, w1)

    def _group(g, carry):
        row0 = sid * CPT + g * G
        pltpu.sync_copy(dst_hbm.at[pl.ds(row0, G)], dstb)
        pltpu.sync_copy(src_hbm.at[pl.ds(row0, G)], sgb)
        pltpu.sync_copy(ea_hbm.at[pl.ds(row0, G)], eab)
        for gg in range(G):
            for t in range(128 // 16):
                sl = pl.ds(t * 16, 16)
                dgb[gg, sl] = dstb[gg, sl] * 2 + cid
                sgb[gg, sl] = sgb[gg, sl] * 2 + cid
        for k in range(G):
            cpa = pltpu.async_copy(a_hbm.at[dgb.at[k]], abuf, sem_a)
            cpb = pltpu.async_copy(b_hbm.at[sgb.at[k]], bbuf, sem_b)
            cpa.wait()
            cpb.wait()

            def _edge_blk(blk, icarry):
                ev = eab[k, pl.ds(blk * 16, 16)]
                for j in range(16):
                    i = blk * 16 + j
                    e = ev[j]
                    for r in range(2):
                        a = abuf[i, pl.ds(r * 16, 16)]
                        b = bbuf[i, pl.ds(r * 16, 16)]
                        obuf[i, pl.ds(r * 16, 16)] = jnp.maximum(
                            a + b + e * wes[r], 0.0)
                return icarry
            lax.fori_loop(0, 8, _edge_blk, 0)
            pltpu.sync_copy(obuf, acc.at[dstb.at[k]], add=True)

            @pl.when(cid == 0)
            def _():
                pltpu.sync_copy(dbuf, dacc.at[dstb.at[k]], add=True)
        return carry
    lax.fori_loop(0, GPT, _group, 0)

    plsc.subcore_barrier()
    row = sid * ROWS_PER_TILE
    pltpu.sync_copy(acc.at[pl.ds(row, ROWS_PER_TILE)],
                    out_hbm.at[cid, pl.ds(row, ROWS_PER_TILE)])

    @pl.when(cid == 0)
    def _():
        pltpu.sync_copy(dacc.at[pl.ds(row, ROWS_PER_TILE)],
                        deg_hbm.at[pl.ds(row, ROWS_PER_TILE)])


def kernel(constraint_features, edge_index, edge_attr, variable_features, params):
    p = params
    cf = jnp.pad(constraint_features, ((0, N_PAD - N), (0, 0)))
    vf = jnp.pad(variable_features, ((0, N_PAD - N), (0, 0)))
    ei0 = jnp.pad(edge_index[0], (0, E_PAD - E), constant_values=N).reshape(EROWS, 128)
    ei1 = jnp.pad(edge_index[1], (0, E_PAD - E), constant_values=N).reshape(EROWS, 128)
    eas = jnp.pad(edge_attr[:, 0], (0, E_PAD - E)).reshape(EROWS, 128)

    c0, v0, avc, bvc, acv = _t1(cf, vf, p)
    s1, deg_c = _edge_pass(avc.reshape(2 * N_PAD, HALF), bvc.reshape(2 * N_PAD, HALF),
                           ei0, ei1, eas, p['vc_We'][:, 0])
    bcv = _t2(s1[0], s1[1], deg_c, c0, p)
    s2, deg_v = _edge_pass(acv.reshape(2 * N_PAD, HALF), bcv.reshape(2 * N_PAD, HALF),
                           ei1, ei0, eas, p['cv_We'][:, 0])
    value, logits = _t3(s2[0], s2[1], deg_v, v0, p)
    return value[:N, 0] + p['bv2'][0], logits[:N, 0]


# speedup vs baseline: 7.8259x; 1.6374x over previous
import jax
import jax.numpy as jnp
from jax import lax
from jax.experimental import pallas as pl
from jax.experimental.pallas import tpu as pltpu
from jax.experimental.pallas import tpu_sc as plsc

EMB = 64
HALF = 32
N = 25000
E = 800000
N_PAD = 25088
E_PAD = 802816
NC, NS = 2, 16
EROWS = E_PAD // 128
CPT = EROWS // NS
G = 8
GPT = CPT // G
ROWS_PER_TILE = N_PAD // NS
ZROWS = 98

BLK = 512
GRID = N_PAD // BLK


def _dotT(x, w):
    return lax.dot_general(x.astype(jnp.bfloat16), w.astype(jnp.bfloat16),
                           (((1,), (1,)), ((), ())),
                           preferred_element_type=jnp.float32)


def _bf16r(x):
    u = lax.bitcast_convert_type(x, jnp.uint32)
    u = ((u + jnp.uint32(0x7FFF) + ((u >> 16) & jnp.uint32(1)))
         & jnp.uint32(0xFFFF0000))
    return lax.bitcast_convert_type(u, jnp.float32)


def _dotT_1(x, w):
    return _dotT_hi(_bf16r(x), _bf16r(w))


def _dotT_hi(x, w):
    return lax.dot_general(x, w, (((1,), (1,)), ((), ())),
                           precision=lax.Precision.HIGHEST,
                           preferred_element_type=jnp.float32)


def _relu(x):
    return jnp.maximum(x, 0.0)


def _row_spec(d):
    return pl.BlockSpec((BLK, d), lambda i: (i, 0))


def _w_spec(shape):
    return pl.BlockSpec(shape, lambda i: tuple(0 for _ in shape))


def _t1_body(cf, vf, Wce1, bce1, Wce2, bce2, Wve1, bve1, Wve2, bve2,
             vcWl, vcbl, vcWr, cvWl, cvbl,
             c0o, v0o, avco, bvco, acvo):
    c0 = _relu(_dotT(_relu(_dotT(cf[...], Wce1[...]) + bce1[...]), Wce2[...]) + bce2[...])
    v0 = _relu(_dotT(_relu(_dotT(vf[...], Wve1[...]) + bve1[...]), Wve2[...]) + bve2[...])
    c0o[...] = c0
    v0o[...] = v0
    avco[...] = _dotT(c0, vcWl[...]) + vcbl[...]
    bvco[...] = _dotT(v0, vcWr[...])
    acvo[...] = _dotT(v0, cvWl[...]) + cvbl[...]


def _t1(cf, vf, p):
    w = [p['Wce1'], p['bce1'].reshape(1, -1), p['Wce2'], p['bce2'].reshape(1, -1),
         p['Wve1'], p['bve1'].reshape(1, -1), p['Wve2'], p['bve2'].reshape(1, -1),
         p['vc_Wl'], p['vc_bl'].reshape(1, -1), p['vc_Wr'],
         p['cv_Wl'], p['cv_bl'].reshape(1, -1)]
    out_shape = [jax.ShapeDtypeStruct((N_PAD, EMB), jnp.float32)] * 5
    return pl.pallas_call(
        _t1_body,
        grid=(GRID,),
        in_specs=[_row_spec(cf.shape[1]), _row_spec(vf.shape[1])] + [_w_spec(x.shape) for x in w],
        out_specs=[_row_spec(EMB)] * 5,
        out_shape=out_shape,
    )(cf, vf, *w)


def _t2_body(slo, shi, deg, c0, Wfl, Wfh, bf, Wo1, bo1, Wo2, bo2, cvWr,
             bcvo):
    agg = (_dotT_hi(slo[...], _bf16r(Wfl[...])) + _dotT_hi(shi[...], _bf16r(Wfh[...]))
           + deg[:, :1] * bf[...])
    cat = jnp.concatenate([agg, c0[...]], axis=-1)
    t = _relu(_dotT(cat, Wo1[...]) + bo1[...])
    c1 = _dotT(t, Wo2[...]) + bo2[...]
    bcvo[...] = _dotT(c1, cvWr[...])


def _t2(slo, shi, deg, c0, p):
    w = [p['vc_Wf'][:, :HALF], p['vc_Wf'][:, HALF:], p['vc_bf'].reshape(1, -1),
         p['vc_Wo1'], p['vc_bo1'].reshape(1, -1),
         p['vc_Wo2'], p['vc_bo2'].reshape(1, -1), p['cv_Wr']]
    return pl.pallas_call(
        _t2_body,
        grid=(GRID,),
        in_specs=[_row_spec(HALF), _row_spec(HALF), _row_spec(16), _row_spec(EMB)]
                 + [_w_spec(x.shape) for x in w],
        out_specs=_row_spec(EMB),
        out_shape=jax.ShapeDtypeStruct((N_PAD, EMB), jnp.float32),
    )(slo, shi, deg, c0, *w)


def _t3_body(slo, shi, deg, v0, Wfl, Wfh, bf, Wo1, bo1, Wo2, bo2,
             Wout1, bout1, Wout2, Wv1, bv1, Wv2, valo, lgo):
    agg = (_dotT_hi(slo[...], _bf16r(Wfl[...])) + _dotT_hi(shi[...], _bf16r(Wfh[...]))
           + deg[:, :1] * bf[...])
    cat = jnp.concatenate([agg, v0[...]], axis=-1)
    t = _relu(_dotT(cat, Wo1[...]) + bo1[...])
    v1 = _dotT(t, Wo2[...]) + bo2[...]
    lgo[...] = _dotT_1(_relu(_dotT(v1, Wout1[...]) + bout1[...]), Wout2[...])
    valo[...] = _dotT_1(_relu(_dotT(v1, Wv1[...]) + bv1[...]), Wv2[...])


def _t3(slo, shi, deg, v0, p):
    w = [p['cv_Wf'][:, :HALF], p['cv_Wf'][:, HALF:], p['cv_bf'].reshape(1, -1),
         p['cv_Wo1'], p['cv_bo1'].reshape(1, -1),
         p['cv_Wo2'], p['cv_bo2'].reshape(1, -1),
         p['Wout1'], p['bout1'].reshape(1, -1), p['Wout2'],
         p['Wv1'], p['bv1'].reshape(1, -1), p['Wv2']]
    return pl.pallas_call(
        _t3_body,
        grid=(GRID,),
        in_specs=[_row_spec(HALF), _row_spec(HALF), _row_spec(16), _row_spec(EMB)]
                 + [_w_spec(x.shape) for x in w],
        out_specs=[_row_spec(1), _row_spec(1)],
        out_shape=[jax.ShapeDtypeStruct((N_PAD, 1), jnp.float32),
                   jax.ShapeDtypeStruct((N_PAD, 1), jnp.float32)],
    )(slo, shi, deg, v0, *w)


def _make_edge_pass(do_deg):
    out_type = [jax.ShapeDtypeStruct((NC, N_PAD, HALF), jnp.float32)]
    if do_deg:
        out_type.append(jax.ShapeDtypeStruct((NC, N_PAD, 16), jnp.float32))
    scratch = [
        pltpu.VMEM((2, G, 128), jnp.int32),
        pltpu.VMEM((2, G, 128), jnp.int32),
        pltpu.VMEM((2, G, 128), jnp.int32),
        pltpu.VMEM((2, G, 128), jnp.int32),
        pltpu.VMEM((2, G, 128), jnp.float32),
        pltpu.VMEM((2, 128, HALF), jnp.float32),
        pltpu.VMEM((2, 128, HALF), jnp.float32),
        pltpu.VMEM((2, 128, HALF), jnp.float32),
        pltpu.VMEM((128, 16), jnp.float32),
        pltpu.VMEM((EMB,), jnp.float32),
        pltpu.VMEM_SHARED((N_PAD, HALF), jnp.float32),
        pltpu.SemaphoreType.DMA,
        pltpu.SemaphoreType.DMA,
        pltpu.SemaphoreType.DMA,
        pltpu.SemaphoreType.DMA,
        pltpu.SemaphoreType.DMA,
        pltpu.SemaphoreType.DMA,
        pltpu.SemaphoreType.DMA,
    ]
    if do_deg:
        scratch.append(pltpu.VMEM_SHARED((N_PAD, 16), jnp.float32))
        scratch.append(pltpu.SemaphoreType.DMA)

    def body(a_hbm, b_hbm, dst_hbm, src_hbm, ea_hbm, we_hbm, out_hbm, *rest):
        if do_deg:
            deg_hbm, rest = rest[0], rest[1:]
        (dstb, srcb, dgb, sgb, eab, abuf, bbuf, obuf, dbuf, webuf, acc,
         sem_i, sem_a0, sem_a1, sem_b0, sem_b1, sem_s0, sem_s1) = rest[:18]
        if do_deg:
            dacc, sem_d = rest[18], rest[19]
        sem_a = (sem_a0, sem_a1)
        sem_b = (sem_b0, sem_b1)
        sem_s = (sem_s0, sem_s1)

        cid = lax.axis_index("c")
        sid = lax.axis_index("s")
        pltpu.sync_copy(we_hbm, webuf)

        zero16 = jnp.zeros((16,), jnp.float32)

        def _zrow(i, carry):
            for r in range(HALF // 16):
                obuf[0, i, pl.ds(r * 16, 16)] = zero16
            if do_deg:
                dbuf[i, pl.ds(0, 16)] = zero16
            return carry
        lax.fori_loop(0, 128, _zrow, 0)

        def _zstripe(k, carry):
            row = sid * ROWS_PER_TILE + k * ZROWS
            pltpu.sync_copy(obuf.at[0, pl.ds(0, ZROWS)], acc.at[pl.ds(row, ZROWS)])
            if do_deg:
                pltpu.sync_copy(dbuf.at[pl.ds(0, ZROWS)], dacc.at[pl.ds(row, ZROWS)])
            return carry
        lax.fori_loop(0, ROWS_PER_TILE // ZROWS, _zstripe, 0)

        if do_deg:
            onehot = jnp.where(lax.iota(jnp.int32, 16) == 0, 1.0, 0.0)

            def _orow(i, carry):
                dbuf[i, pl.ds(0, 16)] = onehot
                return carry
            lax.fori_loop(0, 128, _orow, 0)
        plsc.subcore_barrier()

        w0 = jnp.where(cid == 0, webuf[pl.ds(0, 16)], webuf[pl.ds(32, 16)])
        w1 = jnp.where(cid == 0, webuf[pl.ds(16, 16)], webuf[pl.ds(48, 16)])
        wes = (w0, w1)

        def stage(g, slot):
            row0 = sid * CPT + g * G
            pltpu.async_copy(dst_hbm.at[pl.ds(row0, G)], dstb.at[slot], sem_i)
            pltpu.async_copy(src_hbm.at[pl.ds(row0, G)], srcb.at[slot], sem_i)
            pltpu.async_copy(ea_hbm.at[pl.ds(row0, G)], eab.at[slot], sem_i)

        def wait_stage(slot):
            pltpu.make_async_copy(dst_hbm.at[pl.ds(0, G)], dstb.at[slot], sem_i).wait()
            pltpu.make_async_copy(src_hbm.at[pl.ds(0, G)], srcb.at[slot], sem_i).wait()
            pltpu.make_async_copy(ea_hbm.at[pl.ds(0, G)], eab.at[slot], sem_i).wait()

        def transform(slot):
            for gg in range(G):
                for t in range(128 // 16):
                    sl = pl.ds(t * 16, 16)
                    dgb[slot, gg, sl] = dstb[slot, gg, sl] * 2 + cid
                    sgb[slot, gg, sl] = srcb[slot, gg, sl] * 2 + cid

        def start_gather(slot, kk, q):
            pltpu.async_copy(a_hbm.at[dgb.at[slot, kk]], abuf.at[q], sem_a[q])
            pltpu.async_copy(b_hbm.at[sgb.at[slot, kk]], bbuf.at[q], sem_b[q])

        def wait_gather(q):
            pltpu.make_async_copy(a_hbm.at[dgb.at[0, 0]], abuf.at[q], sem_a[q]).wait()
            pltpu.make_async_copy(b_hbm.at[sgb.at[0, 0]], bbuf.at[q], sem_b[q]).wait()

        def start_scatter(slot, kk, q):
            pltpu.async_copy(obuf.at[q], acc.at[dstb.at[slot, kk]], sem_s[q], add=True)

        def wait_scatter(q):
            pltpu.make_async_copy(obuf.at[q], acc.at[dstb.at[0, 0]], sem_s[q]).wait()

        def start_deg(slot, kk):
            @pl.when(cid == 0)
            def _():
                pltpu.async_copy(dbuf, dacc.at[dstb.at[slot, kk]], sem_d, add=True)

            @pl.when(cid == 1)
            def _():
                pltpu.async_copy(dbuf, dacc.at[srcb.at[slot, kk]], sem_d, add=True)

        def wait_deg():
            pltpu.make_async_copy(dbuf, dacc.at[dstb.at[0, 0]], sem_d).wait()

        def compute(slot, kk, q):
            def _edge_blk(blk, icarry):
                ev = eab[slot, kk, pl.ds(blk * 16, 16)]
                for j in range(16):
                    i = blk * 16 + j
                    e = ev[j]
                    for r in range(2):
                        a = abuf[q, i, pl.ds(r * 16, 16)]
                        b = bbuf[q, i, pl.ds(r * 16, 16)]
                        x = jnp.maximum((a + e * wes[r]) + b, 0.0)
                        u = lax.bitcast_convert_type(x, jnp.uint32)
                        u = ((u + jnp.uint32(0x7FFF) + ((u >> 16) & jnp.uint32(1)))
                             & jnp.uint32(0xFFFF0000))
                        obuf[q, i, pl.ds(r * 16, 16)] = (
                            lax.bitcast_convert_type(u, jnp.float32))
                return icarry
            lax.fori_loop(0, 8, _edge_blk, 0)

        def prep_next(slot, q_last):
            wait_stage(1 - slot)
            transform(1 - slot)
            start_gather(1 - slot, 0, 1 - q_last)

        def do_group(g, slot, first, guard_next):
            for k in range(G):
                q = k % 2
                wait_gather(q)
                if k < G - 1:
                    start_gather(slot, k + 1, 1 - q)
                if not (first and k < 2):
                    wait_scatter(q)
                compute(slot, k, q)
                if do_deg:
                    if not (first and k == 0):
                        wait_deg()
                    start_deg(slot, k)
                start_scatter(slot, k, q)
                if k == 1:
                    if guard_next:
                        @pl.when(g + 1 < GPT)
                        def _():
                            stage(g + 1, 1 - slot)
                    else:
                        stage(g + 1, 1 - slot)
                if k == G - 1:
                    if guard_next:
                        @pl.when(g + 1 < GPT)
                        def _():
                            prep_next(slot, q)
                    else:
                        prep_next(slot, q)

        stage(0, 0)
        wait_stage(0)
        transform(0)
        start_gather(0, 0, 0)
        do_group(0, 0, first=True, guard_next=False)

        def pair_body(i, carry):
            g0 = 1 + 2 * i
            do_group(g0, 1, first=False, guard_next=False)
            do_group(g0 + 1, 0, first=False, guard_next=True)
            return carry
        lax.fori_loop(0, (GPT - 1) // 2, pair_body, 0)

        wait_scatter(0)
        wait_scatter(1)
        if do_deg:
            wait_deg()

        plsc.subcore_barrier()
        row = sid * ROWS_PER_TILE
        pltpu.sync_copy(acc.at[pl.ds(row, ROWS_PER_TILE)],
                        out_hbm.at[cid, pl.ds(row, ROWS_PER_TILE)])
        if do_deg:
            pltpu.sync_copy(dacc.at[pl.ds(row, ROWS_PER_TILE)],
                            deg_hbm.at[cid, pl.ds(row, ROWS_PER_TILE)])

    return pl.kernel(
        body,
        out_type=out_type if do_deg else out_type[0],
        mesh=plsc.VectorSubcoreMesh(core_axis_name="c", subcore_axis_name="s"),
        compiler_params=pltpu.CompilerParams(use_tc_tiling_on_sc=False),
        scratch_types=scratch,
    )


_edge_pass_deg = _make_edge_pass(True)
_edge_pass_nodeg = _make_edge_pass(False)


def kernel(constraint_features, edge_index, edge_attr, variable_features, params):
    p = params
    cf = jnp.pad(constraint_features, ((0, N_PAD - N), (0, 0)))
    vf = jnp.pad(variable_features, ((0, N_PAD - N), (0, 0)))
    ei0 = jnp.pad(edge_index[0], (0, E_PAD - E), constant_values=N).reshape(EROWS, 128)
    ei1 = jnp.pad(edge_index[1], (0, E_PAD - E), constant_values=N).reshape(EROWS, 128)
    eas = jnp.pad(edge_attr[:, 0], (0, E_PAD - E)).reshape(EROWS, 128)

    c0, v0, avc, bvc, acv = _t1(cf, vf, p)
    s1, degs = _edge_pass_deg(avc.reshape(2 * N_PAD, HALF), bvc.reshape(2 * N_PAD, HALF),
                              ei0, ei1, eas, p['vc_We'][:, 0])
    bcv = _t2(s1[0], s1[1], degs[0], c0, p)
    s2 = _edge_pass_nodeg(acv.reshape(2 * N_PAD, HALF), bcv.reshape(2 * N_PAD, HALF),
                          ei1, ei0, eas, p['cv_We'][:, 0])
    value, logits = _t3(s2[0], s2[1], degs[1], v0, p)
    return value[:N, 0] + p['bv2'][0], logits[:N, 0]


# speedup vs baseline: 7.8352x; 1.0012x over previous
import jax
import jax.numpy as jnp
from jax import lax
from jax.experimental import pallas as pl
from jax.experimental.pallas import tpu as pltpu
from jax.experimental.pallas import tpu_sc as plsc

EMB = 64
HALF = 32
N = 25000
E = 800000
N_PAD = 25088
E_PAD = 802816
NC, NS = 2, 16
EROWS = E_PAD // 128
CPT = EROWS // NS
G = 8
GPT = CPT // G
ROWS_PER_TILE = N_PAD // NS
ZROWS = 98

BLK = 512
GRID = N_PAD // BLK


def _dotT(x, w):
    return lax.dot_general(x.astype(jnp.bfloat16), w.astype(jnp.bfloat16),
                           (((1,), (1,)), ((), ())),
                           preferred_element_type=jnp.float32)


def _bf16r(x):
    u = lax.bitcast_convert_type(x, jnp.uint32)
    u = ((u + jnp.uint32(0x7FFF) + ((u >> 16) & jnp.uint32(1)))
         & jnp.uint32(0xFFFF0000))
    return lax.bitcast_convert_type(u, jnp.float32)


def _dotT_1(x, w):
    return _dotT_hi(_bf16r(x), _bf16r(w))


def _dotT_hi(x, w):
    return lax.dot_general(x, w, (((1,), (1,)), ((), ())),
                           precision=lax.Precision.HIGHEST,
                           preferred_element_type=jnp.float32)


def _relu(x):
    return jnp.maximum(x, 0.0)


def _row_spec(d):
    return pl.BlockSpec((BLK, d), lambda i: (i, 0))


def _w_spec(shape):
    return pl.BlockSpec(shape, lambda i: tuple(0 for _ in shape))


def _t1_body(cf, vf, Wce1, bce1, Wce2, bce2, Wve1, bve1, Wve2, bve2,
             vcWl, vcbl, vcWr, cvWl, cvbl,
             c0o, v0o, avco, bvco, acvo):
    c0 = _relu(_dotT(_relu(_dotT(cf[...], Wce1[...]) + bce1[...]), Wce2[...]) + bce2[...])
    v0 = _relu(_dotT(_relu(_dotT(vf[...], Wve1[...]) + bve1[...]), Wve2[...]) + bve2[...])
    c0o[...] = c0
    v0o[...] = v0
    avco[...] = _dotT(c0, vcWl[...]) + vcbl[...]
    bvco[...] = _dotT(v0, vcWr[...])
    acvo[...] = _dotT(v0, cvWl[...]) + cvbl[...]


def _t1(cf, vf, p):
    w = [p['Wce1'], p['bce1'].reshape(1, -1), p['Wce2'], p['bce2'].reshape(1, -1),
         p['Wve1'], p['bve1'].reshape(1, -1), p['Wve2'], p['bve2'].reshape(1, -1),
         p['vc_Wl'], p['vc_bl'].reshape(1, -1), p['vc_Wr'],
         p['cv_Wl'], p['cv_bl'].reshape(1, -1)]
    out_shape = [jax.ShapeDtypeStruct((N_PAD, EMB), jnp.float32)] * 5
    return pl.pallas_call(
        _t1_body,
        grid=(GRID,),
        in_specs=[_row_spec(cf.shape[1]), _row_spec(vf.shape[1])] + [_w_spec(x.shape) for x in w],
        out_specs=[_row_spec(EMB)] * 5,
        out_shape=out_shape,
    )(cf, vf, *w)


def _t2_body(slo, shi, deg, c0, Wfl, Wfh, bf, Wo1, bo1, Wo2, bo2, cvWr,
             bcvo):
    agg = (_dotT_hi(slo[...], _bf16r(Wfl[...])) + _dotT_hi(shi[...], _bf16r(Wfh[...]))
           + deg[:, :1] * bf[...])
    cat = jnp.concatenate([agg, c0[...]], axis=-1)
    t = _relu(_dotT(cat, Wo1[...]) + bo1[...])
    c1 = _dotT(t, Wo2[...]) + bo2[...]
    bcvo[...] = _dotT(c1, cvWr[...])


def _t2(slo, shi, deg, c0, p):
    w = [p['vc_Wf'][:, :HALF], p['vc_Wf'][:, HALF:], p['vc_bf'].reshape(1, -1),
         p['vc_Wo1'], p['vc_bo1'].reshape(1, -1),
         p['vc_Wo2'], p['vc_bo2'].reshape(1, -1), p['cv_Wr']]
    return pl.pallas_call(
        _t2_body,
        grid=(GRID,),
        in_specs=[_row_spec(HALF), _row_spec(HALF), _row_spec(16), _row_spec(EMB)]
                 + [_w_spec(x.shape) for x in w],
        out_specs=_row_spec(EMB),
        out_shape=jax.ShapeDtypeStruct((N_PAD, EMB), jnp.float32),
    )(slo, shi, deg, c0, *w)


def _t3_body(slo, shi, deg, v0, Wfl, Wfh, bf, Wo1, bo1, Wo2, bo2,
             Wout1, bout1, Wout2, Wv1, bv1, Wv2, valo, lgo):
    agg = (_dotT_hi(slo[...], _bf16r(Wfl[...])) + _dotT_hi(shi[...], _bf16r(Wfh[...]))
           + deg[:, :1] * bf[...])
    cat = jnp.concatenate([agg, v0[...]], axis=-1)
    t = _relu(_dotT(cat, Wo1[...]) + bo1[...])
    v1 = _dotT(t, Wo2[...]) + bo2[...]
    lgo[...] = _dotT_1(_relu(_dotT(v1, Wout1[...]) + bout1[...]), Wout2[...])
    valo[...] = _dotT_1(_relu(_dotT(v1, Wv1[...]) + bv1[...]), Wv2[...])


def _t3(slo, shi, deg, v0, p):
    w = [p['cv_Wf'][:, :HALF], p['cv_Wf'][:, HALF:], p['cv_bf'].reshape(1, -1),
         p['cv_Wo1'], p['cv_bo1'].reshape(1, -1),
         p['cv_Wo2'], p['cv_bo2'].reshape(1, -1),
         p['Wout1'], p['bout1'].reshape(1, -1), p['Wout2'],
         p['Wv1'], p['bv1'].reshape(1, -1), p['Wv2']]
    return pl.pallas_call(
        _t3_body,
        grid=(GRID,),
        in_specs=[_row_spec(HALF), _row_spec(HALF), _row_spec(16), _row_spec(EMB)]
                 + [_w_spec(x.shape) for x in w],
        out_specs=[_row_spec(1), _row_spec(1)],
        out_shape=[jax.ShapeDtypeStruct((N_PAD, 1), jnp.float32),
                   jax.ShapeDtypeStruct((N_PAD, 1), jnp.float32)],
    )(slo, shi, deg, v0, *w)


def _make_edge_pass(do_deg):
    out_type = [jax.ShapeDtypeStruct((NC, N_PAD, HALF), jnp.float32)]
    if do_deg:
        out_type.append(jax.ShapeDtypeStruct((NC, N_PAD, 16), jnp.float32))
    scratch = [
        pltpu.VMEM((2, G, 128), jnp.int32),
        pltpu.VMEM((2, G, 128), jnp.int32),
        pltpu.VMEM((2, G, 128), jnp.int32),
        pltpu.VMEM((2, G, 128), jnp.int32),
        pltpu.VMEM((2, G, 128), jnp.float32),
        pltpu.VMEM((2, 128, HALF), jnp.float32),
        pltpu.VMEM((2, 128, HALF), jnp.float32),
        pltpu.VMEM((2, 128, HALF), jnp.float32),
        pltpu.VMEM((128, 16), jnp.float32),
        pltpu.VMEM((EMB,), jnp.float32),
        pltpu.VMEM_SHARED((N_PAD, HALF), jnp.float32),
        pltpu.SemaphoreType.DMA,
        pltpu.SemaphoreType.DMA,
        pltpu.SemaphoreType.DMA,
        pltpu.SemaphoreType.DMA,
        pltpu.SemaphoreType.DMA,
        pltpu.SemaphoreType.DMA,
        pltpu.SemaphoreType.DMA,
    ]
    if do_deg:
        scratch.append(pltpu.VMEM_SHARED((N_PAD, 16), jnp.float32))
        scratch.append(pltpu.SemaphoreType.DMA)

    def body(a_hbm, b_hbm, dst_hbm, src_hbm, ea_hbm, we_hbm, out_hbm, *rest):
        if do_deg:
            deg_hbm, rest = rest[0], rest[1:]
        (dstb, srcb, dgb, sgb, eab, abuf, bbuf, obuf, dbuf, webuf, acc,
         sem_i, sem_a0, sem_a1, sem_b0, sem_b1, sem_s0, sem_s1) = rest[:18]
        if do_deg:
            dacc, sem_d = rest[18], rest[19]
        sem_a = (sem_a0, sem_a1)
        sem_b = (sem_b0, sem_b1)
        sem_s = (sem_s0, sem_s1)

        cid = lax.axis_index("c")
        sid = lax.axis_index("s")
        pltpu.sync_copy(we_hbm, webuf)

        zero16 = jnp.zeros((16,), jnp.float32)

        def _zrow(i, carry):
            for r in range(HALF // 16):
                obuf[0, i, pl.ds(r * 16, 16)] = zero16
            if do_deg:
                dbuf[i, pl.ds(0, 16)] = zero16
            return carry
        lax.fori_loop(0, 128, _zrow, 0)

        def _zstripe(k, carry):
            row = sid * ROWS_PER_TILE + k * ZROWS
            pltpu.sync_copy(obuf.at[0, pl.ds(0, ZROWS)], acc.at[pl.ds(row, ZROWS)])
            if do_deg:
                pltpu.sync_copy(dbuf.at[pl.ds(0, ZROWS)], dacc.at[pl.ds(row, ZROWS)])
            return carry
        lax.fori_loop(0, ROWS_PER_TILE // ZROWS, _zstripe, 0)

        if do_deg:
            onehot = jnp.where(lax.iota(jnp.int32, 16) == 0, 1.0, 0.0)

            def _orow(i, carry):
                dbuf[i, pl.ds(0, 16)] = onehot
                return carry
            lax.fori_loop(0, 128, _orow, 0)
        plsc.subcore_barrier()

        w0 = jnp.where(cid == 0, webuf[pl.ds(0, 16)], webuf[pl.ds(32, 16)])
        w1 = jnp.where(cid == 0, webuf[pl.ds(16, 16)], webuf[pl.ds(48, 16)])
        wes = (w0, w1)

        def stage(g, slot):
            row0 = sid * CPT + g * G
            pltpu.async_copy(dst_hbm.at[pl.ds(row0, G)], dstb.at[slot], sem_i)
            pltpu.async_copy(src_hbm.at[pl.ds(row0, G)], srcb.at[slot], sem_i)
            pltpu.async_copy(ea_hbm.at[pl.ds(row0, G)], eab.at[slot], sem_i)

        def wait_stage(slot):
            pltpu.make_async_copy(dst_hbm.at[pl.ds(0, G)], dstb.at[slot], sem_i).wait()
            pltpu.make_async_copy(src_hbm.at[pl.ds(0, G)], srcb.at[slot], sem_i).wait()
            pltpu.make_async_copy(ea_hbm.at[pl.ds(0, G)], eab.at[slot], sem_i).wait()

        def transform(slot):
            for gg in range(G):
                for t in range(128 // 16):
                    sl = pl.ds(t * 16, 16)
                    dgb[slot, gg, sl] = dstb[slot, gg, sl] * 2 + cid
                    sgb[slot, gg, sl] = srcb[slot, gg, sl] * 2 + cid

        def start_gather(slot, kk, q):
            pltpu.async_copy(a_hbm.at[dgb.at[slot, kk]], abuf.at[q], sem_a[q])
            pltpu.async_copy(b_hbm.at[sgb.at[slot, kk]], bbuf.at[q], sem_b[q])

        def wait_gather(q):
            pltpu.make_async_copy(a_hbm.at[dgb.at[0, 0]], abuf.at[q], sem_a[q]).wait()
            pltpu.make_async_copy(b_hbm.at[sgb.at[0, 0]], bbuf.at[q], sem_b[q]).wait()

        def start_scatter(slot, kk, q):
            pltpu.async_copy(obuf.at[q], acc.at[dstb.at[slot, kk]], sem_s[q], add=True)

        def wait_scatter(q):
            pltpu.make_async_copy(obuf.at[q], acc.at[dstb.at[0, 0]], sem_s[q]).wait()

        def start_deg(slot, kk):
            @pl.when(cid == 0)
            def _():
                pltpu.async_copy(dbuf, dacc.at[dstb.at[slot, kk]], sem_d, add=True)

            @pl.when(cid == 1)
            def _():
                pltpu.async_copy(dbuf, dacc.at[srcb.at[slot, kk]], sem_d, add=True)

        def wait_deg():
            pltpu.make_async_copy(dbuf, dacc.at[dstb.at[0, 0]], sem_d).wait()

        def compute(slot, kk, q):
            def _edge_blk(blk, icarry):
                ev = eab[slot, kk, pl.ds(blk * 16, 16)]
                for j in range(16):
                    i = blk * 16 + j
                    e = ev[j]
                    for r in range(2):
                        a = abuf[q, i, pl.ds(r * 16, 16)]
                        b = bbuf[q, i, pl.ds(r * 16, 16)]
                        x = jnp.maximum((a + e * wes[r]) + b, 0.0)
                        u = lax.bitcast_convert_type(x, jnp.uint32)
                        u = ((u + jnp.uint32(0x7FFF) + ((u >> 16) & jnp.uint32(1)))
                             & jnp.uint32(0xFFFF0000))
                        obuf[q, i, pl.ds(r * 16, 16)] = (
                            lax.bitcast_convert_type(u, jnp.float32))
                return icarry
            lax.fori_loop(0, 8, _edge_blk, 0)

        def prep_next(slot, q_last):
            wait_stage(1 - slot)
            transform(1 - slot)
            start_gather(1 - slot, 0, 1 - q_last)

        def do_group(g, slot, first, guard_next):
            for k in range(G):
                q = k % 2
                wait_gather(q)
                if k < G - 1:
                    start_gather(slot, k + 1, 1 - q)
                if not (first and k < 2):
                    wait_scatter(q)
                compute(slot, k, q)
                if do_deg:
                    if not (first and k == 0):
                        wait_deg()
                    start_deg(slot, k)
                start_scatter(slot, k, q)
                if k == 1:
                    if guard_next:
                        @pl.when(g + 1 < GPT)
                        def _():
                            stage(g + 1, 1 - slot)
                    else:
                        stage(g + 1, 1 - slot)
                if k == G - 1:
                    if guard_next:
                        @pl.when(g + 1 < GPT)
                        def _():
                            prep_next(slot, q)
                    else:
                        prep_next(slot, q)

        stage(0, 0)
        wait_stage(0)
        transform(0)
        start_gather(0, 0, 0)
        do_group(0, 0, first=True, guard_next=False)

        def pair_body(i, carry):
            g0 = 1 + 2 * i
            do_group(g0, 1, first=False, guard_next=False)
            do_group(g0 + 1, 0, first=False, guard_next=True)
            return carry
        lax.fori_loop(0, (GPT - 1) // 2, pair_body, 0)

        wait_scatter(0)
        wait_scatter(1)
        if do_deg:
            wait_deg()

        plsc.subcore_barrier()
        row = sid * ROWS_PER_TILE
        pltpu.sync_copy(acc.at[pl.ds(row, ROWS_PER_TILE)],
                        out_hbm.at[cid, pl.ds(row, ROWS_PER_TILE)])
        if do_deg:
            pltpu.sync_copy(dacc.at[pl.ds(row, ROWS_PER_TILE)],
                            deg_hbm.at[cid, pl.ds(row, ROWS_PER_TILE)])

    return pl.kernel(
        body,
        out_type=out_type if do_deg else out_type[0],
        mesh=plsc.VectorSubcoreMesh(core_axis_name="c", subcore_axis_name="s",
                                    num_cores=NC, num_subcores=NS),
        compiler_params=pltpu.CompilerParams(use_tc_tiling_on_sc=False),
        scratch_types=scratch,
    )


_EDGE_PASS_CACHE = {}


def _edge_pass(do_deg, *args):
    fn = _EDGE_PASS_CACHE.get(do_deg)
    if fn is None:
        fn = _make_edge_pass(do_deg)
        _EDGE_PASS_CACHE[do_deg] = fn
    return fn(*args)


def kernel(constraint_features, edge_index, edge_attr, variable_features, params):
    p = params
    cf = jnp.pad(constraint_features, ((0, N_PAD - N), (0, 0)))
    vf = jnp.pad(variable_features, ((0, N_PAD - N), (0, 0)))
    ei0 = jnp.pad(edge_index[0], (0, E_PAD - E), constant_values=N).reshape(EROWS, 128)
    ei1 = jnp.pad(edge_index[1], (0, E_PAD - E), constant_values=N).reshape(EROWS, 128)
    eas = jnp.pad(edge_attr[:, 0], (0, E_PAD - E)).reshape(EROWS, 128)

    c0, v0, avc, bvc, acv = _t1(cf, vf, p)
    s1, degs = _edge_pass(True, avc.reshape(2 * N_PAD, HALF), bvc.reshape(2 * N_PAD, HALF),
                          ei0, ei1, eas, p['vc_We'][:, 0])
    bcv = _t2(s1[0], s1[1], degs[0], c0, p)
    s2 = _edge_pass(False, acv.reshape(2 * N_PAD, HALF), bcv.reshape(2 * N_PAD, HALF),
                    ei1, ei0, eas, p['cv_We'][:, 0])
    value, logits = _t3(s2[0], s2[1], degs[1], v0, p)
    return value[:N, 0] + p['bv2'][0], logits[:N, 0]
